# baseline (speedup 1.0000x reference)
import jax
import jax.numpy as jnp
from jax.experimental import pallas as pl


def _masked_kernel(x_ref, tpT_ref, masked_ref, x_out_ref, tmpl_ref):
    xb = x_ref[0]
    hw, c = xb.shape
    tpT = tpT_ref[...]
    mx = jnp.max(xb, axis=0)
    iota = jax.lax.broadcasted_iota(jnp.int32, (hw, c), 0)
    idx = jnp.min(jnp.where(xb == mx[None, :], iota, hw), axis=0)
    idx = jnp.where(mx == 0.0, hw - 1, idx)
    onehot = (iota == idx[None, :]).astype(jnp.float32)
    tmpl = jax.lax.dot_general(
        tpT, onehot, (((1,), (0,)), ((), ())),
        preferred_element_type=jnp.float32)
    tmpl_ref[0] = tmpl
    x_out_ref[0] = xb
    masked_ref[0] = jnp.maximum(xb * tmpl, 0.0)


def kernel(x, t_p):
    b, h, w, c = x.shape
    hw = h * w
    xr = jnp.reshape(x, (b, hw, c))
    tpT = jnp.transpose(jnp.reshape(t_p, (hw, hw)), (1, 0))
    masked_r, x_out_r, tmpl_r = pl.pallas_call(
        _masked_kernel,
        grid=(b,),
        in_specs=[
            pl.BlockSpec((1, hw, c), lambda i: (i, 0, 0)),
            pl.BlockSpec((hw, hw), lambda i: (0, 0)),
        ],
        out_specs=[
            pl.BlockSpec((1, hw, c), lambda i: (i, 0, 0)),
            pl.BlockSpec((1, hw, c), lambda i: (i, 0, 0)),
            pl.BlockSpec((1, hw, c), lambda i: (i, 0, 0)),
        ],
        out_shape=[
            jax.ShapeDtypeStruct((b, hw, c), jnp.float32),
            jax.ShapeDtypeStruct((b, hw, c), jnp.float32),
            jax.ShapeDtypeStruct((b, hw, c), jnp.float32),
        ],
    )(xr, tpT)
    masked = jnp.reshape(masked_r, (b, h, w, c))
    x_out = jnp.reshape(x_out_r, (b, h, w, c))
    templates = jnp.reshape(tmpl_r, (b, h, w, c))
    return (masked, x_out, templates)


# speedup vs baseline: 1.2660x; 1.2660x over previous
import jax
import jax.numpy as jnp
from jax.experimental import pallas as pl
from jax.experimental.pallas import tpu as pltpu

_NB = 4


def _masked_kernel(x_ref, tpT_ref, masked_ref, tmpl_ref):
    tpT = tpT_ref[...]
    for n in range(_NB):
        xb = x_ref[n]
        hw, c = xb.shape
        mx = jnp.max(xb, axis=0)
        iota = jax.lax.broadcasted_iota(jnp.int32, (hw, c), 0)
        idx = jnp.min(jnp.where(xb == mx[None, :], iota, hw), axis=0)
        idx = jnp.where(mx == 0.0, hw - 1, idx)
        onehot = (iota == idx[None, :]).astype(jnp.float32)
        tmpl = jax.lax.dot_general(
            tpT, onehot, (((1,), (0,)), ((), ())),
            preferred_element_type=jnp.float32)
        tmpl_ref[n] = tmpl
        masked_ref[n] = jnp.maximum(xb * tmpl, 0.0)


def kernel(x, t_p):
    b, h, w, c = x.shape
    hw = h * w
    xr = jnp.reshape(x, (b, hw, c))
    tpT = jnp.transpose(jnp.reshape(t_p, (hw, hw)), (1, 0))
    masked_r, tmpl_r = pl.pallas_call(
        _masked_kernel,
        grid=(b // _NB,),
        in_specs=[
            pl.BlockSpec((_NB, hw, c), lambda i: (i, 0, 0)),
            pl.BlockSpec((hw, hw), lambda i: (0, 0)),
        ],
        out_specs=[
            pl.BlockSpec((_NB, hw, c), lambda i: (i, 0, 0)),
            pl.BlockSpec((_NB, hw, c), lambda i: (i, 0, 0)),
        ],
        out_shape=[
            jax.ShapeDtypeStruct((b, hw, c), jnp.float32),
            jax.ShapeDtypeStruct((b, hw, c), jnp.float32),
        ],
        compiler_params=pltpu.CompilerParams(
            dimension_semantics=("arbitrary",),
        ),
    )(xr, tpT)
    masked = jnp.reshape(masked_r, (b, h, w, c))
    templates = jnp.reshape(tmpl_r, (b, h, w, c))
    return (masked, x, templates)


# speedup vs baseline: 1.2836x; 1.0140x over previous
import jax
import jax.numpy as jnp
from jax.experimental import pallas as pl
from jax.experimental.pallas import tpu as pltpu

_NB = 8


def _masked_kernel(x_ref, tpT_ref, masked_ref, tmpl_ref):
    tpT = tpT_ref[...]
    for n in range(_NB):
        xb = x_ref[n]
        hw, c = xb.shape
        mx = jnp.max(xb, axis=0)
        iota = jax.lax.broadcasted_iota(jnp.int32, (hw, c), 0)
        idx = jnp.min(jnp.where(xb == mx[None, :], iota, hw), axis=0)
        idx = jnp.where(mx == 0.0, hw - 1, idx)
        onehot = (iota == idx[None, :]).astype(jnp.float32)
        tmpl = jax.lax.dot_general(
            tpT, onehot, (((1,), (0,)), ((), ())),
            preferred_element_type=jnp.float32)
        tmpl_ref[n] = tmpl
        masked_ref[n] = jnp.maximum(xb * tmpl, 0.0)


def kernel(x, t_p):
    b, h, w, c = x.shape
    hw = h * w
    xr = jnp.reshape(x, (b, hw, c))
    tpT = jnp.transpose(jnp.reshape(t_p, (hw, hw)), (1, 0))
    masked_r, tmpl_r = pl.pallas_call(
        _masked_kernel,
        grid=(b // _NB,),
        in_specs=[
            pl.BlockSpec((_NB, hw, c), lambda i: (i, 0, 0)),
            pl.BlockSpec((hw, hw), lambda i: (0, 0)),
        ],
        out_specs=[
            pl.BlockSpec((_NB, hw, c), lambda i: (i, 0, 0)),
            pl.BlockSpec((_NB, hw, c), lambda i: (i, 0, 0)),
        ],
        out_shape=[
            jax.ShapeDtypeStruct((b, hw, c), jnp.float32),
            jax.ShapeDtypeStruct((b, hw, c), jnp.float32),
        ],
        compiler_params=pltpu.CompilerParams(
            dimension_semantics=("arbitrary",),
        ),
    )(xr, tpT)
    masked = jnp.reshape(masked_r, (b, h, w, c))
    templates = jnp.reshape(tmpl_r, (b, h, w, c))
    return (masked, x, templates)


# speedup vs baseline: 1.2906x; 1.0054x over previous
import jax
import jax.numpy as jnp
from jax.experimental import pallas as pl
from jax.experimental.pallas import tpu as pltpu

_NB = 16


def _masked_kernel(x_ref, tpT_ref, masked_ref, tmpl_ref):
    tpT = tpT_ref[...]
    for n in range(_NB):
        xb = x_ref[n]
        hw, c = xb.shape
        mx = jnp.max(xb, axis=0)
        iota = jax.lax.broadcasted_iota(jnp.int32, (hw, c), 0)
        idx = jnp.min(jnp.where(xb == mx[None, :], iota, hw), axis=0)
        idx = jnp.where(mx == 0.0, hw - 1, idx)
        onehot = (iota == idx[None, :]).astype(jnp.float32)
        tmpl = jax.lax.dot_general(
            tpT, onehot, (((1,), (0,)), ((), ())),
            preferred_element_type=jnp.float32)
        tmpl_ref[n] = tmpl
        masked_ref[n] = jnp.maximum(xb * tmpl, 0.0)


def kernel(x, t_p):
    b, h, w, c = x.shape
    hw = h * w
    xr = jnp.reshape(x, (b, hw, c))
    tpT = jnp.transpose(jnp.reshape(t_p, (hw, hw)), (1, 0))
    masked_r, tmpl_r = pl.pallas_call(
        _masked_kernel,
        grid=(b // _NB,),
        in_specs=[
            pl.BlockSpec((_NB, hw, c), lambda i: (i, 0, 0)),
            pl.BlockSpec((hw, hw), lambda i: (0, 0)),
        ],
        out_specs=[
            pl.BlockSpec((_NB, hw, c), lambda i: (i, 0, 0)),
            pl.BlockSpec((_NB, hw, c), lambda i: (i, 0, 0)),
        ],
        out_shape=[
            jax.ShapeDtypeStruct((b, hw, c), jnp.float32),
            jax.ShapeDtypeStruct((b, hw, c), jnp.float32),
        ],
        compiler_params=pltpu.CompilerParams(
            dimension_semantics=("parallel",),
        ),
    )(xr, tpT)
    masked = jnp.reshape(masked_r, (b, h, w, c))
    templates = jnp.reshape(tmpl_r, (b, h, w, c))
    return (masked, x, templates)
